# baseline (speedup 1.0000x reference)
import functools

import jax
import jax.numpy as jnp
from jax import lax
from jax.experimental import pallas as pl
from jax.experimental.pallas import tpu as pltpu
from jax.experimental.pallas import tpu_sc as plsc

N = 10000
E = 160000
D_IN = 256
HID = 32
NUM_CLASSES = 4

NPAD = 10240
NW, NC, L = 32, 2, 16
BLK = 256
GRID = NPAD // BLK


def _seg_kernel(F, TW, NS, NPB, P, K, G, with_cnt, probe=False):
    assert E % K == 0 and K % L == 0 and G == L and F % L == 0
    assert NW * NPB * P == NPAD
    NCH = E // K
    assert NCH % 2 == 0
    KP = K + L
    NF = F // L
    TWS = TW // NS
    NFS = TWS // L
    mesh = plsc.VectorSubcoreMesh(core_axis_name="c", subcore_axis_name="s",
                                  num_cores=NC, num_subcores=NW // NC)
    out_type = [jax.ShapeDtypeStruct((NPAD, F), jnp.float32) for _ in range(4)]
    if with_cnt:
        out_type.append(jax.ShapeDtypeStruct((NPAD, L), jnp.float32))
    scratch = [
        pltpu.VMEM((2, K), jnp.int32),
        pltpu.VMEM((2, K), jnp.int32),
        pltpu.VMEM((2, K), jnp.float32),
        pltpu.VMEM((2, K), jnp.float32),
        pltpu.VMEM((KP,), jnp.int32),
        pltpu.VMEM((KP,), jnp.int32),
        pltpu.VMEM((KP,), jnp.float32),
        pltpu.VMEM((KP,), jnp.float32),
    ]
    scratch += [pltpu.VMEM((4 * G, TWS), jnp.float32)
                for _ in range(NS)]
    scratch += [
        pltpu.VMEM((8, TW), jnp.float32),
        pltpu.VMEM((NPB + 1, F), jnp.float32),
        pltpu.VMEM((NPB + 1, F), jnp.float32),
        pltpu.VMEM((NPB + 1, F), jnp.float32),
        pltpu.VMEM((NPB + 1, F), jnp.float32),
    ]
    if with_cnt:
        scratch.append(pltpu.VMEM((NPB + 1, L), jnp.float32))
    scratch += [pltpu.SemaphoreType.DMA] * 6

    def body(dst_h, src_h, a_h, b_h, *args):
        T_hs = args[:NS]
        C_h = args[NS]
        refs = args[NS + 1:]
        if with_cnt:
            sum_h, ssq_h, mn_h, mx_h, cnt_h = refs[:5]
            refs = refs[5:]
            (dst_v, src_v, a_v, b_v, selL, selS, selA, selB, *rows_l, C_v,
             acc_s, acc_q, acc_n, acc_x, acc_c,
             se0, se1, sg0, sg1, sg2, sg3) = refs
        else:
            sum_h, ssq_h, mn_h, mx_h = refs[:4]
            cnt_h = None
            refs = refs[4:]
            (dst_v, src_v, a_v, b_v, selL, selS, selA, selB, *rows_l, C_v,
             acc_s, acc_q, acc_n, acc_x,
             se0, se1, sg0, sg1, sg2, sg3) = refs
            acc_c = None
        sems_e = (se0, se1)
        sems_g = (sg0, sg1, sg2, sg3)
        edge_hv = ((dst_h, dst_v), (src_h, src_v), (a_h, a_v), (b_h, b_v))
        wid = lax.axis_index("s") * NC + lax.axis_index("c")
        pltpu.sync_copy(C_h, C_v)
        zero = jnp.zeros((L,), jnp.float32)
        lane = lax.broadcasted_iota(jnp.int32, (L,), 0)
        one0 = jnp.where(lane == 0, 1.0, 0.0).astype(jnp.float32)

        def _initsel(i, _):
            selS[pl.ds(i * L, L)] = jnp.zeros((L,), jnp.int32)
            return 0
        lax.fori_loop(0, KP // L, _initsel, 0)

        def _issue_edges(c, buf):
            for hb, vb in edge_hv:
                pltpu.async_copy(hb.at[pl.ds(c * K, K)], vb.at[buf],
                                 sems_e[buf])

        def _wait_edges(c, buf):
            for hb, vb in edge_hv:
                pltpu.make_async_copy(hb.at[pl.ds(c * K, K)], vb.at[buf],
                                      sems_e[buf]).wait()

        class _GatherSet:
            def __init__(self, copies):
                self.copies = copies

            def start(self):
                for cp in self.copies:
                    cp.start()

            def wait(self):
                for cp in self.copies:
                    cp.wait()

        def _gather(base, buf):
            return _GatherSet([
                pltpu.make_async_copy(
                    T_hs[t].at[selS.at[pl.ds(base, G)]],
                    rows_l[t].at[pl.ds(buf * G, G)], sems_g[buf])
                for t in range(NS)])

        def run_pass(p, _):
            lo = (p * NW + wid) * NPB
            _issue_edges(0, 0)

            def _initacc(i, _):
                for f in range(NF):
                    s = pl.ds(f * L, L)
                    acc_s[i, s] = zero
                    acc_q[i, s] = zero
                    acc_n[i, s] = jnp.full((L,), jnp.inf, jnp.float32)
                    acc_x[i, s] = jnp.full((L,), -jnp.inf, jnp.float32)
                if with_cnt:
                    acc_c[i, pl.ds(0, L)] = zero
                return 0
            lax.fori_loop(0, NPB, _initacc, 0)

            def chunk_fn(c, _):
                par = c & 1

                @pl.when(par == 0)
                def _():
                    _wait_edges(c, 0)

                @pl.when(par == 1)
                def _():
                    _wait_edges(c, 1)

                @pl.when(c + 1 < NCH)
                def _():
                    @pl.when(par == 0)
                    def _():
                        _issue_edges(c + 1, 1)

                    @pl.when(par == 1)
                    def _():
                        _issue_edges(c + 1, 0)

                def scan(j, count):
                    s = pl.ds(j * L, L)
                    locv = dst_v[par, s] - lo
                    msk = (locv >= 0) & (locv < NPB)
                    w = pl.ds(count, L)
                    plsc.store_compressed(selL.at[w], locv, mask=msk)
                    plsc.store_compressed(selS.at[w], src_v[par, s], mask=msk)
                    plsc.store_compressed(selA.at[w], a_v[par, s], mask=msk)
                    plsc.store_compressed(selB.at[w], b_v[par, s], mask=msk)
                    return count + plsc.all_reduce_population_count(msk)[0]
                count = lax.fori_loop(0, K // L, scan, jnp.int32(0))
                w = pl.ds(count, L)
                selL[w] = jnp.full((L,), NPB, jnp.int32)
                selS[w] = jnp.zeros((L,), jnp.int32)
                selA[w] = zero
                selB[w] = zero
                nb = (count + G - 1) // G

                for q in range(3):
                    @pl.when(q < nb)
                    def _(q=q):
                        _gather(q * G, q).start()

                def batch(gb, _):
                    p2 = gb & 3
                    base = gb * G
                    for q in range(4):
                        @pl.when(p2 == q)
                        def _(q=q):
                            _gather(base, q).wait()

                    @pl.when(gb + 3 < nb)
                    def _():
                        pn = (gb + 3) & 3
                        for q in range(4):
                            @pl.when(pn == q)
                            def _(q=q):
                                _gather(base + 3 * G, q).start()

                    rbase = p2 * G
                    locv = selL[pl.ds(base, L)]
                    av = selA[pl.ds(base, L)]
                    bv = selB[pl.ds(base, L)]
                    for j in (() if probe else range(L)):
                        loc = locv[j]
                        a = av[j]
                        b = bv[j]
                        r = rbase + j
                        for f in range(NF):
                            s = pl.ds(f * L, L)
                            sl = pl.ds((f % NFS) * L, L)
                            g = (rows_l[f // NFS][r, sl] + a * C_v[0, s]
                                 + b * C_v[1, s])
                            plsc.addupdate(acc_s.at[loc, s], g)
                            plsc.addupdate(acc_q.at[loc, s], g * g)
                            acc_n[loc, s] = jnp.minimum(acc_n[loc, s], g)
                            acc_x[loc, s] = jnp.maximum(acc_x[loc, s], g)
                        if with_cnt:
                            plsc.addupdate(acc_c.at[loc, pl.ds(0, L)], one0)
                    return 0
                lax.fori_loop(0, nb, batch, 0)
                return 0
            lax.fori_loop(0, NCH, chunk_fn, 0)
            r = pl.ds(lo, NPB)
            pltpu.sync_copy(acc_s.at[pl.ds(0, NPB)], sum_h.at[r])
            pltpu.sync_copy(acc_q.at[pl.ds(0, NPB)], ssq_h.at[r])
            pltpu.sync_copy(acc_n.at[pl.ds(0, NPB)], mn_h.at[r])
            pltpu.sync_copy(acc_x.at[pl.ds(0, NPB)], mx_h.at[r])
            if with_cnt:
                pltpu.sync_copy(acc_c.at[pl.ds(0, NPB)], cnt_h.at[r])
            return 0
        lax.fori_loop(0, P, run_pass, 0)

    return pl.kernel(body, out_type=out_type, mesh=mesh, scratch_types=scratch,
                     compiler_params=pltpu.CompilerParams(
                         needs_layout_passes=False,
                         use_tc_tiling_on_sc=False))


_seg1 = _seg_kernel(F=D_IN, TW=D_IN, NS=2, NPB=80, P=4, K=2000, G=16, with_cnt=True)
_seg2 = _seg_kernel(F=HID, TW=128, NS=1, NPB=320, P=1, K=2000, G=16, with_cnt=False)


def _mm_body(x_ref, w_ref, b_ref, o_ref):
    o_ref[...] = (jnp.dot(x_ref[...], w_ref[...],
                          preferred_element_type=jnp.float32) + b_ref[...])


def _project(x, W, b):
    Fin, Fout = W.shape
    return pl.pallas_call(
        _mm_body,
        grid=(GRID,),
        in_specs=[
            pl.BlockSpec((BLK, Fin), lambda i: (i, 0)),
            pl.BlockSpec((Fin, Fout), lambda i: (0, 0)),
            pl.BlockSpec((1, Fout), lambda i: (0, 0)),
        ],
        out_specs=pl.BlockSpec((BLK, Fout), lambda i: (i, 0)),
        out_shape=jax.ShapeDtypeStruct((NPAD, Fout), jnp.float32),
    )(x, W, b.reshape(1, Fout))


def _avg_log_body(cnt_ref, o_ref):
    i = pl.program_id(0)
    c = cnt_ref[:, 0:1]
    rid = jax.lax.broadcasted_iota(jnp.int32, (BLK, 1), 0) + i * BLK
    val = jnp.where(rid < N, jnp.log(c + 1.0), 0.0)
    s = jnp.sum(val)

    @pl.when(i == 0)
    def _():
        o_ref[0, 0] = 0.0
    o_ref[0, 0] += s


def _avg_log(cnt):
    out = pl.pallas_call(
        _avg_log_body,
        grid=(GRID,),
        in_specs=[pl.BlockSpec((BLK, L), lambda i: (i, 0))],
        out_specs=pl.BlockSpec(memory_space=pltpu.SMEM),
        out_shape=jax.ShapeDtypeStruct((1, 1), jnp.float32),
    )(cnt)
    return out / N


def _combine_body(F, sum_ref, ssq_ref, mn_ref, mx_ref, cnt_ref, a_ref, x_ref,
                  avl_ref, qx_ref, qa_ref, qb_ref, qc_ref, qbias_ref,
                  lw_ref, lb_ref, o_ref):
    cnt = cnt_ref[:, 0:1]
    cntc = jnp.maximum(cnt, 1.0)
    A = a_ref[...]
    sg = sum_ref[...]
    sum_h = sg + cnt * A
    mean = sum_h / cntc
    msq = (ssq_ref[...] + 2.0 * A * sg + cnt * A * A) / cntc
    std = jnp.sqrt(jax.nn.relu(msq - mean * mean) + 1e-5)
    mask = cnt > 0.0
    mn = jnp.where(mask, mn_ref[...] + A, 0.0)
    mx = jnp.where(mask, mx_ref[...] + A, 0.0)
    agg = jnp.concatenate([mean, mn, mx, std], axis=1)
    avl = avl_ref[0, 0]
    lg = jnp.log(cntc + 1.0)
    amp = lg / avl
    att = avl / lg
    out = (jnp.dot(x_ref[...], qx_ref[...], preferred_element_type=jnp.float32)
           + jnp.dot(agg, qa_ref[...], preferred_element_type=jnp.float32)
           + jnp.dot(agg * amp, qb_ref[...], preferred_element_type=jnp.float32)
           + jnp.dot(agg * att, qc_ref[...], preferred_element_type=jnp.float32)
           + qbias_ref[...])
    o_ref[...] = (jnp.dot(out, lw_ref[...], preferred_element_type=jnp.float32)
                  + lb_ref[...])


def _combine(F, Fout, SUM, SSQ, MN, MX, CNT, A, Xin, avl, Qx, Qa, Qb, Qc, qb,
             lW, lb):
    Fx = Xin.shape[1]
    Fmid = Qx.shape[1]
    return pl.pallas_call(
        functools.partial(_combine_body, F),
        grid=(GRID,),
        in_specs=[
            pl.BlockSpec((BLK, F), lambda i: (i, 0)),
            pl.BlockSpec((BLK, F), lambda i: (i, 0)),
            pl.BlockSpec((BLK, F), lambda i: (i, 0)),
            pl.BlockSpec((BLK, F), lambda i: (i, 0)),
            pl.BlockSpec((BLK, L), lambda i: (i, 0)),
            pl.BlockSpec((BLK, F), lambda i: (i, 0)),
            pl.BlockSpec((BLK, Fx), lambda i: (i, 0)),
            pl.BlockSpec(memory_space=pltpu.SMEM),
            pl.BlockSpec((Fx, Fmid), lambda i: (0, 0)),
            pl.BlockSpec((4 * F, Fmid), lambda i: (0, 0)),
            pl.BlockSpec((4 * F, Fmid), lambda i: (0, 0)),
            pl.BlockSpec((4 * F, Fmid), lambda i: (0, 0)),
            pl.BlockSpec((1, Fmid), lambda i: (0, 0)),
            pl.BlockSpec((Fmid, Fout), lambda i: (0, 0)),
            pl.BlockSpec((1, Fout), lambda i: (0, 0)),
        ],
        out_specs=pl.BlockSpec((BLK, Fout), lambda i: (i, 0)),
        out_shape=jax.ShapeDtypeStruct((NPAD, Fout), jnp.float32),
    )(SUM, SSQ, MN, MX, CNT, A, Xin, avl, Qx, Qa, Qb, Qc,
      qb.reshape(1, Fmid), lW, lb.reshape(1, Fout))


def _bn_stats_body(h_ref, o_ref):
    i = pl.program_id(0)
    h = h_ref[...]
    rid = jax.lax.broadcasted_iota(jnp.int32, (BLK, 1), 0) + i * BLK
    hm = jnp.where(rid < N, h, 0.0)
    s = jnp.sum(hm, axis=0, keepdims=True)
    q = jnp.sum(hm * hm, axis=0, keepdims=True)

    @pl.when(i == 0)
    def _():
        o_ref[...] = jnp.zeros_like(o_ref)
    o_ref[0:1, :] += s
    o_ref[1:2, :] += q


def _bn_stats(h):
    return pl.pallas_call(
        _bn_stats_body,
        grid=(GRID,),
        in_specs=[pl.BlockSpec((BLK, HID), lambda i: (i, 0))],
        out_specs=pl.BlockSpec((2, HID), lambda i: (0, 0)),
        out_shape=jax.ShapeDtypeStruct((2, HID), jnp.float32),
    )(h)


def _bn_act_body(h_ref, st_ref, g_ref, b_ref, w_ref, c_ref, act_ref, ab_ref):
    h = h_ref[...]
    m = st_ref[0:1, :] / N
    v = st_ref[1:2, :] / N - m * m
    hn = g_ref[...] * (h - m) / jnp.sqrt(v + 1e-5) + b_ref[...]
    act = jnp.where(hn > 0.0, hn, jnp.exp(hn) - 1.0)
    act_ref[...] = act
    ab_ref[...] = (jnp.dot(act, w_ref[...], preferred_element_type=jnp.float32)
                   + c_ref[...])


def _bn_act(h, stats, bn_g, bn_b, W2, c2):
    return pl.pallas_call(
        _bn_act_body,
        grid=(GRID,),
        in_specs=[
            pl.BlockSpec((BLK, HID), lambda i: (i, 0)),
            pl.BlockSpec((2, HID), lambda i: (0, 0)),
            pl.BlockSpec((1, HID), lambda i: (0, 0)),
            pl.BlockSpec((1, HID), lambda i: (0, 0)),
            pl.BlockSpec((HID, HID + 128), lambda i: (0, 0)),
            pl.BlockSpec((1, HID + 128), lambda i: (0, 0)),
        ],
        out_specs=[
            pl.BlockSpec((BLK, HID), lambda i: (i, 0)),
            pl.BlockSpec((BLK, HID + 128), lambda i: (i, 0)),
        ],
        out_shape=[
            jax.ShapeDtypeStruct((NPAD, HID), jnp.float32),
            jax.ShapeDtypeStruct((NPAD, HID + 128), jnp.float32),
        ],
    )(h, stats, bn_g.reshape(1, HID), bn_b.reshape(1, HID), W2,
      c2.reshape(1, HID + 128))


def _logsm_body(z_ref, o_ref):
    z = z_ref[...]
    zmax = jnp.max(z, axis=1, keepdims=True)
    ez = jnp.exp(z - zmax)
    lse = jnp.log(jnp.sum(ez, axis=1, keepdims=True))
    o_ref[...] = z - zmax - lse


def _logsm(z):
    return pl.pallas_call(
        _logsm_body,
        grid=(GRID,),
        in_specs=[pl.BlockSpec((BLK, NUM_CLASSES), lambda i: (i, 0))],
        out_specs=pl.BlockSpec((BLK, NUM_CLASSES), lambda i: (i, 0)),
        out_shape=jax.ShapeDtypeStruct((NPAD, NUM_CLASSES), jnp.float32),
    )(z)


def kernel(x, edge_index, edge_attr, eW1, eb1, pW1, pb1, qW1, qb1, lW1, lb1,
           bn_g, bn_b, eW2, eb2, pW2, pb2, qW2, qb2, lW2, lb2):
    f32 = jnp.float32
    src = edge_index[0].astype(jnp.int32)
    dst = edge_index[1].astype(jnp.int32)
    ea = edge_attr[:, 0].astype(f32)
    eb = edge_attr[:, 1].astype(f32)

    Wd1, Ws1, We1 = pW1[:D_IN], pW1[D_IN:2 * D_IN], pW1[2 * D_IN:]
    C1 = jnp.concatenate([eW1 @ We1, jnp.zeros((6, D_IN), f32)])
    c01 = eb1 @ We1 + pb1
    W1 = jnp.concatenate([Wd1, Ws1], axis=1)
    bias1 = jnp.concatenate([c01, jnp.zeros((D_IN,), f32)])
    Wd2, Ws2, We2 = pW2[:HID], pW2[HID:2 * HID], pW2[2 * HID:]
    C2 = jnp.concatenate([eW2 @ We2, jnp.zeros((6, HID), f32)],
                         axis=0)
    C2 = jnp.concatenate([C2, jnp.zeros((8, 128 - HID), f32)], axis=1)
    c02 = eb2 @ We2 + pb2
    Ws2p = jnp.concatenate([Ws2, jnp.zeros((HID, 128 - HID), f32)], axis=1)
    W2 = jnp.concatenate([Wd2, Ws2p], axis=1)
    bias2 = jnp.concatenate([c02, jnp.zeros((128,), f32)])
    Qx1, Qa1 = qW1[:D_IN], qW1[D_IN:5 * D_IN]
    Qb1, Qc1 = qW1[5 * D_IN:9 * D_IN], qW1[9 * D_IN:]
    Qx2, Qa2 = qW2[:HID], qW2[HID:5 * HID]
    Qb2, Qc2 = qW2[5 * HID:9 * HID], qW2[9 * HID:]

    x_pad = jnp.zeros((NPAD, D_IN), f32).at[:N].set(x)

    AB1 = _project(x_pad, W1, bias1)
    A1, B1 = AB1[:, :D_IN], AB1[:, D_IN:]
    B1a, B1b = AB1[:, D_IN:D_IN + 128], AB1[:, D_IN + 128:]
    S1, Q1s, MN1, MX1, CNT = _seg1(dst, src, ea, eb, B1a, B1b, C1)
    avl = _avg_log(CNT)
    h = _combine(D_IN, HID, S1, Q1s, MN1, MX1, CNT, A1, x_pad, avl,
                 Qx1, Qa1, Qb1, Qc1, qb1, lW1, lb1)
    stats = _bn_stats(h)
    act, AB2 = _bn_act(h, stats, bn_g, bn_b, W2, bias2)
    A2, B2 = AB2[:, :HID], AB2[:, HID:]
    S2, Q2s, MN2, MX2 = _seg2(dst, src, ea, eb, B2, C2)
    z = _combine(HID, NUM_CLASSES, S2, Q2s, MN2, MX2, CNT, A2, act, avl,
                 Qx2, Qa2, Qb2, Qc2, qb2, lW2, lb2)
    out = _logsm(z)
    return out[:N]


# speedup vs baseline: 1.2190x; 1.2190x over previous
import functools

import jax
import jax.numpy as jnp
from jax import lax
from jax.experimental import pallas as pl
from jax.experimental.pallas import tpu as pltpu
from jax.experimental.pallas import tpu_sc as plsc

N = 10000
E = 160000
D_IN = 256
HID = 32
NUM_CLASSES = 4

NPAD = 10240
NW, NC, L = 32, 2, 16
BLK = 256
GRID = NPAD // BLK


def _seg_kernel(F, TW, NS, NPB, P, K, G, with_cnt, probe=False):
    assert E % K == 0 and K % L == 0 and G == L and F % L == 0
    assert NW * NPB * P == NPAD
    NCH = E // K
    assert NCH % 2 == 0
    KP = K + L
    NF = F // L
    TWS = TW // NS
    NFS = TWS // L
    mesh = plsc.VectorSubcoreMesh(core_axis_name="c", subcore_axis_name="s",
                                  num_cores=NC, num_subcores=NW // NC)
    out_type = [jax.ShapeDtypeStruct((NPAD, F), jnp.float32) for _ in range(4)]
    if with_cnt:
        out_type.append(jax.ShapeDtypeStruct((NPAD, L), jnp.float32))
    scratch = [
        pltpu.VMEM((2, K), jnp.int32),
        pltpu.VMEM((2, K), jnp.int32),
        pltpu.VMEM((2, K), jnp.float32),
        pltpu.VMEM((2, K), jnp.float32),
        pltpu.VMEM((KP,), jnp.int32),
        pltpu.VMEM((KP,), jnp.int32),
        pltpu.VMEM((KP,), jnp.float32),
        pltpu.VMEM((KP,), jnp.float32),
    ]
    scratch += [pltpu.VMEM((4 * G, TWS), jnp.float32)
                for _ in range(NS)]
    scratch += [
        pltpu.VMEM((8, TW), jnp.float32),
        pltpu.VMEM((NPB + 1, F), jnp.float32),
        pltpu.VMEM((NPB + 1, F), jnp.float32),
        pltpu.VMEM((NPB + 1, F), jnp.float32),
        pltpu.VMEM((NPB + 1, F), jnp.float32),
    ]
    if with_cnt:
        scratch.append(pltpu.VMEM((NPB + 1, L), jnp.float32))
    scratch += [pltpu.SemaphoreType.DMA] * 6

    def body(dst_h, src_h, a_h, b_h, *args):
        T_hs = args[:NS]
        C_h = args[NS]
        refs = args[NS + 1:]
        if with_cnt:
            sum_h, ssq_h, mn_h, mx_h, cnt_h = refs[:5]
            refs = refs[5:]
            (dst_v, src_v, a_v, b_v, selL, selS, selA, selB, *rows_l, C_v,
             acc_s, acc_q, acc_n, acc_x, acc_c,
             se0, se1, sg0, sg1, sg2, sg3) = refs
        else:
            sum_h, ssq_h, mn_h, mx_h = refs[:4]
            cnt_h = None
            refs = refs[4:]
            (dst_v, src_v, a_v, b_v, selL, selS, selA, selB, *rows_l, C_v,
             acc_s, acc_q, acc_n, acc_x,
             se0, se1, sg0, sg1, sg2, sg3) = refs
            acc_c = None
        sems_e = (se0, se1)
        sems_g = (sg0, sg1, sg2, sg3)
        edge_hv = ((dst_h, dst_v), (src_h, src_v), (a_h, a_v), (b_h, b_v))
        wid = lax.axis_index("s") * NC + lax.axis_index("c")
        pltpu.sync_copy(C_h, C_v)
        zero = jnp.zeros((L,), jnp.float32)
        lane = lax.broadcasted_iota(jnp.int32, (L,), 0)
        one0 = jnp.where(lane == 0, 1.0, 0.0).astype(jnp.float32)

        def _initsel(i, _):
            selS[pl.ds(i * L, L)] = jnp.zeros((L,), jnp.int32)
            return 0
        lax.fori_loop(0, KP // L, _initsel, 0)

        def _issue_edges(c, buf):
            for hb, vb in edge_hv:
                pltpu.async_copy(hb.at[pl.ds(c * K, K)], vb.at[buf],
                                 sems_e[buf])

        def _wait_edges(c, buf):
            for hb, vb in edge_hv:
                pltpu.make_async_copy(hb.at[pl.ds(c * K, K)], vb.at[buf],
                                      sems_e[buf]).wait()

        class _GatherSet:
            def __init__(self, copies):
                self.copies = copies

            def start(self):
                for cp in self.copies:
                    cp.start()

            def wait(self):
                for cp in self.copies:
                    cp.wait()

        def _gather(base, buf):
            return _GatherSet([
                pltpu.make_async_copy(
                    T_hs[t].at[selS.at[pl.ds(base, G)]],
                    rows_l[t].at[pl.ds(buf * G, G)], sems_g[buf])
                for t in range(NS)])

        def run_pass(p, _):
            lo = (p * NW + wid) * NPB
            _issue_edges(0, 0)

            def _initacc(i, _):
                for f in range(NF):
                    s = pl.ds(f * L, L)
                    acc_s[i, s] = zero
                    acc_q[i, s] = zero
                    acc_n[i, s] = jnp.full((L,), jnp.inf, jnp.float32)
                    acc_x[i, s] = jnp.full((L,), -jnp.inf, jnp.float32)
                if with_cnt:
                    acc_c[i, pl.ds(0, L)] = zero
                return 0
            lax.fori_loop(0, NPB, _initacc, 0)

            def chunk_fn(c, _):
                par = c & 1

                @pl.when(par == 0)
                def _():
                    _wait_edges(c, 0)

                @pl.when(par == 1)
                def _():
                    _wait_edges(c, 1)

                @pl.when(c + 1 < NCH)
                def _():
                    @pl.when(par == 0)
                    def _():
                        _issue_edges(c + 1, 1)

                    @pl.when(par == 1)
                    def _():
                        _issue_edges(c + 1, 0)

                def scan(j, count):
                    s = pl.ds(j * L, L)
                    locv = dst_v[par, s] - lo
                    msk = (locv >= 0) & (locv < NPB)
                    w = pl.ds(count, L)
                    plsc.store_compressed(selL.at[w], locv, mask=msk)
                    plsc.store_compressed(selS.at[w], src_v[par, s], mask=msk)
                    plsc.store_compressed(selA.at[w], a_v[par, s], mask=msk)
                    plsc.store_compressed(selB.at[w], b_v[par, s], mask=msk)
                    return count + plsc.all_reduce_population_count(msk)[0]
                count = lax.fori_loop(0, K // L, scan, jnp.int32(0))
                w = pl.ds(count, L)
                selL[w] = jnp.full((L,), NPB, jnp.int32)
                selS[w] = jnp.zeros((L,), jnp.int32)
                selA[w] = zero
                selB[w] = zero
                nb = (count + G - 1) // G

                for q in range(3):
                    @pl.when(q < nb)
                    def _(q=q):
                        _gather(q * G, q).start()

                def batch(gb, _):
                    p2 = gb & 3
                    base = gb * G
                    for q in range(4):
                        @pl.when(p2 == q)
                        def _(q=q):
                            _gather(base, q).wait()

                    @pl.when(gb + 3 < nb)
                    def _():
                        pn = (gb + 3) & 3
                        for q in range(4):
                            @pl.when(pn == q)
                            def _(q=q):
                                _gather(base + 3 * G, q).start()

                    rbase = p2 * G
                    locv = selL[pl.ds(base, L)]
                    av = selA[pl.ds(base, L)]
                    bv = selB[pl.ds(base, L)]

                    @functools.partial(plsc.parallel_loop, 0, NF)
                    def _(f):
                        s = pl.ds(f * L, L)
                        for j in (() if probe else range(L)):
                            loc = locv[j]
                            g = (rows_l[0][rbase + j, s] + av[j] * C_v[0, s]
                                 + bv[j] * C_v[1, s])
                            plsc.addupdate(acc_s.at[loc, s], g)
                            plsc.addupdate(acc_q.at[loc, s], g * g)
                            acc_n[loc, s] = jnp.minimum(acc_n[loc, s], g)
                            acc_x[loc, s] = jnp.maximum(acc_x[loc, s], g)
                    if with_cnt and not probe:
                        for j in range(L):
                            plsc.addupdate(acc_c.at[locv[j], pl.ds(0, L)],
                                           one0)
                    return 0
                lax.fori_loop(0, nb, batch, 0)
                return 0
            lax.fori_loop(0, NCH, chunk_fn, 0)
            r = pl.ds(lo, NPB)
            pltpu.sync_copy(acc_s.at[pl.ds(0, NPB)], sum_h.at[r])
            pltpu.sync_copy(acc_q.at[pl.ds(0, NPB)], ssq_h.at[r])
            pltpu.sync_copy(acc_n.at[pl.ds(0, NPB)], mn_h.at[r])
            pltpu.sync_copy(acc_x.at[pl.ds(0, NPB)], mx_h.at[r])
            if with_cnt:
                pltpu.sync_copy(acc_c.at[pl.ds(0, NPB)], cnt_h.at[r])
            return 0
        lax.fori_loop(0, P, run_pass, 0)

    return pl.kernel(body, out_type=out_type, mesh=mesh, scratch_types=scratch,
                     compiler_params=pltpu.CompilerParams(
                         needs_layout_passes=False,
                         use_tc_tiling_on_sc=False))


_seg1 = _seg_kernel(F=D_IN, TW=D_IN, NS=1, NPB=80, P=4, K=2000, G=16, with_cnt=True)
_seg2 = _seg_kernel(F=HID, TW=128, NS=1, NPB=320, P=1, K=2000, G=16, with_cnt=False)


def _mm_body(x_ref, w_ref, b_ref, o_ref):
    o_ref[...] = (jnp.dot(x_ref[...], w_ref[...],
                          preferred_element_type=jnp.float32) + b_ref[...])


def _project(x, W, b):
    Fin, Fout = W.shape
    return pl.pallas_call(
        _mm_body,
        grid=(GRID,),
        in_specs=[
            pl.BlockSpec((BLK, Fin), lambda i: (i, 0)),
            pl.BlockSpec((Fin, Fout), lambda i: (0, 0)),
            pl.BlockSpec((1, Fout), lambda i: (0, 0)),
        ],
        out_specs=pl.BlockSpec((BLK, Fout), lambda i: (i, 0)),
        out_shape=jax.ShapeDtypeStruct((NPAD, Fout), jnp.float32),
    )(x, W, b.reshape(1, Fout))


def _avg_log_body(cnt_ref, o_ref):
    i = pl.program_id(0)
    c = cnt_ref[:, 0:1]
    rid = jax.lax.broadcasted_iota(jnp.int32, (BLK, 1), 0) + i * BLK
    val = jnp.where(rid < N, jnp.log(c + 1.0), 0.0)
    s = jnp.sum(val)

    @pl.when(i == 0)
    def _():
        o_ref[0, 0] = 0.0
    o_ref[0, 0] += s


def _avg_log(cnt):
    out = pl.pallas_call(
        _avg_log_body,
        grid=(GRID,),
        in_specs=[pl.BlockSpec((BLK, L), lambda i: (i, 0))],
        out_specs=pl.BlockSpec(memory_space=pltpu.SMEM),
        out_shape=jax.ShapeDtypeStruct((1, 1), jnp.float32),
    )(cnt)
    return out / N


def _combine_body(F, sum_ref, ssq_ref, mn_ref, mx_ref, cnt_ref, a_ref, x_ref,
                  avl_ref, qx_ref, qa_ref, qb_ref, qc_ref, qbias_ref,
                  lw_ref, lb_ref, o_ref):
    cnt = cnt_ref[:, 0:1]
    cntc = jnp.maximum(cnt, 1.0)
    A = a_ref[...]
    sg = sum_ref[...]
    sum_h = sg + cnt * A
    mean = sum_h / cntc
    msq = (ssq_ref[...] + 2.0 * A * sg + cnt * A * A) / cntc
    std = jnp.sqrt(jax.nn.relu(msq - mean * mean) + 1e-5)
    mask = cnt > 0.0
    mn = jnp.where(mask, mn_ref[...] + A, 0.0)
    mx = jnp.where(mask, mx_ref[...] + A, 0.0)
    agg = jnp.concatenate([mean, mn, mx, std], axis=1)
    avl = avl_ref[0, 0]
    lg = jnp.log(cntc + 1.0)
    amp = lg / avl
    att = avl / lg
    out = (jnp.dot(x_ref[...], qx_ref[...], preferred_element_type=jnp.float32)
           + jnp.dot(agg, qa_ref[...], preferred_element_type=jnp.float32)
           + jnp.dot(agg * amp, qb_ref[...], preferred_element_type=jnp.float32)
           + jnp.dot(agg * att, qc_ref[...], preferred_element_type=jnp.float32)
           + qbias_ref[...])
    o_ref[...] = (jnp.dot(out, lw_ref[...], preferred_element_type=jnp.float32)
                  + lb_ref[...])


def _combine(F, Fout, SUM, SSQ, MN, MX, CNT, A, Xin, avl, Qx, Qa, Qb, Qc, qb,
             lW, lb):
    Fx = Xin.shape[1]
    Fmid = Qx.shape[1]
    return pl.pallas_call(
        functools.partial(_combine_body, F),
        grid=(GRID,),
        in_specs=[
            pl.BlockSpec((BLK, F), lambda i: (i, 0)),
            pl.BlockSpec((BLK, F), lambda i: (i, 0)),
            pl.BlockSpec((BLK, F), lambda i: (i, 0)),
            pl.BlockSpec((BLK, F), lambda i: (i, 0)),
            pl.BlockSpec((BLK, L), lambda i: (i, 0)),
            pl.BlockSpec((BLK, F), lambda i: (i, 0)),
            pl.BlockSpec((BLK, Fx), lambda i: (i, 0)),
            pl.BlockSpec(memory_space=pltpu.SMEM),
            pl.BlockSpec((Fx, Fmid), lambda i: (0, 0)),
            pl.BlockSpec((4 * F, Fmid), lambda i: (0, 0)),
            pl.BlockSpec((4 * F, Fmid), lambda i: (0, 0)),
            pl.BlockSpec((4 * F, Fmid), lambda i: (0, 0)),
            pl.BlockSpec((1, Fmid), lambda i: (0, 0)),
            pl.BlockSpec((Fmid, Fout), lambda i: (0, 0)),
            pl.BlockSpec((1, Fout), lambda i: (0, 0)),
        ],
        out_specs=pl.BlockSpec((BLK, Fout), lambda i: (i, 0)),
        out_shape=jax.ShapeDtypeStruct((NPAD, Fout), jnp.float32),
    )(SUM, SSQ, MN, MX, CNT, A, Xin, avl, Qx, Qa, Qb, Qc,
      qb.reshape(1, Fmid), lW, lb.reshape(1, Fout))


def _bn_stats_body(h_ref, o_ref):
    i = pl.program_id(0)
    h = h_ref[...]
    rid = jax.lax.broadcasted_iota(jnp.int32, (BLK, 1), 0) + i * BLK
    hm = jnp.where(rid < N, h, 0.0)
    s = jnp.sum(hm, axis=0, keepdims=True)
    q = jnp.sum(hm * hm, axis=0, keepdims=True)

    @pl.when(i == 0)
    def _():
        o_ref[...] = jnp.zeros_like(o_ref)
    o_ref[0:1, :] += s
    o_ref[1:2, :] += q


def _bn_stats(h):
    return pl.pallas_call(
        _bn_stats_body,
        grid=(GRID,),
        in_specs=[pl.BlockSpec((BLK, HID), lambda i: (i, 0))],
        out_specs=pl.BlockSpec((2, HID), lambda i: (0, 0)),
        out_shape=jax.ShapeDtypeStruct((2, HID), jnp.float32),
    )(h)


def _bn_act_body(h_ref, st_ref, g_ref, b_ref, w_ref, c_ref, act_ref, ab_ref):
    h = h_ref[...]
    m = st_ref[0:1, :] / N
    v = st_ref[1:2, :] / N - m * m
    hn = g_ref[...] * (h - m) / jnp.sqrt(v + 1e-5) + b_ref[...]
    act = jnp.where(hn > 0.0, hn, jnp.exp(hn) - 1.0)
    act_ref[...] = act
    ab_ref[...] = (jnp.dot(act, w_ref[...], preferred_element_type=jnp.float32)
                   + c_ref[...])


def _bn_act(h, stats, bn_g, bn_b, W2, c2):
    return pl.pallas_call(
        _bn_act_body,
        grid=(GRID,),
        in_specs=[
            pl.BlockSpec((BLK, HID), lambda i: (i, 0)),
            pl.BlockSpec((2, HID), lambda i: (0, 0)),
            pl.BlockSpec((1, HID), lambda i: (0, 0)),
            pl.BlockSpec((1, HID), lambda i: (0, 0)),
            pl.BlockSpec((HID, HID + 128), lambda i: (0, 0)),
            pl.BlockSpec((1, HID + 128), lambda i: (0, 0)),
        ],
        out_specs=[
            pl.BlockSpec((BLK, HID), lambda i: (i, 0)),
            pl.BlockSpec((BLK, HID + 128), lambda i: (i, 0)),
        ],
        out_shape=[
            jax.ShapeDtypeStruct((NPAD, HID), jnp.float32),
            jax.ShapeDtypeStruct((NPAD, HID + 128), jnp.float32),
        ],
    )(h, stats, bn_g.reshape(1, HID), bn_b.reshape(1, HID), W2,
      c2.reshape(1, HID + 128))


def _logsm_body(z_ref, o_ref):
    z = z_ref[...]
    zmax = jnp.max(z, axis=1, keepdims=True)
    ez = jnp.exp(z - zmax)
    lse = jnp.log(jnp.sum(ez, axis=1, keepdims=True))
    o_ref[...] = z - zmax - lse


def _logsm(z):
    return pl.pallas_call(
        _logsm_body,
        grid=(GRID,),
        in_specs=[pl.BlockSpec((BLK, NUM_CLASSES), lambda i: (i, 0))],
        out_specs=pl.BlockSpec((BLK, NUM_CLASSES), lambda i: (i, 0)),
        out_shape=jax.ShapeDtypeStruct((NPAD, NUM_CLASSES), jnp.float32),
    )(z)


def kernel(x, edge_index, edge_attr, eW1, eb1, pW1, pb1, qW1, qb1, lW1, lb1,
           bn_g, bn_b, eW2, eb2, pW2, pb2, qW2, qb2, lW2, lb2):
    f32 = jnp.float32
    src = edge_index[0].astype(jnp.int32)
    dst = edge_index[1].astype(jnp.int32)
    ea = edge_attr[:, 0].astype(f32)
    eb = edge_attr[:, 1].astype(f32)

    Wd1, Ws1, We1 = pW1[:D_IN], pW1[D_IN:2 * D_IN], pW1[2 * D_IN:]
    C1 = jnp.concatenate([eW1 @ We1, jnp.zeros((6, D_IN), f32)])
    c01 = eb1 @ We1 + pb1
    W1 = jnp.concatenate([Wd1, Ws1], axis=1)
    bias1 = jnp.concatenate([c01, jnp.zeros((D_IN,), f32)])
    Wd2, Ws2, We2 = pW2[:HID], pW2[HID:2 * HID], pW2[2 * HID:]
    C2 = jnp.concatenate([eW2 @ We2, jnp.zeros((6, HID), f32)],
                         axis=0)
    C2 = jnp.concatenate([C2, jnp.zeros((8, 128 - HID), f32)], axis=1)
    c02 = eb2 @ We2 + pb2
    Ws2p = jnp.concatenate([Ws2, jnp.zeros((HID, 128 - HID), f32)], axis=1)
    W2 = jnp.concatenate([Wd2, Ws2p], axis=1)
    bias2 = jnp.concatenate([c02, jnp.zeros((128,), f32)])
    Qx1, Qa1 = qW1[:D_IN], qW1[D_IN:5 * D_IN]
    Qb1, Qc1 = qW1[5 * D_IN:9 * D_IN], qW1[9 * D_IN:]
    Qx2, Qa2 = qW2[:HID], qW2[HID:5 * HID]
    Qb2, Qc2 = qW2[5 * HID:9 * HID], qW2[9 * HID:]

    x_pad = jnp.zeros((NPAD, D_IN), f32).at[:N].set(x)

    AB1 = _project(x_pad, W1, bias1)
    A1, B1 = AB1[:, :D_IN], AB1[:, D_IN:]
    S1, Q1s, MN1, MX1, CNT = _seg1(dst, src, ea, eb, B1, C1)
    avl = _avg_log(CNT)
    h = _combine(D_IN, HID, S1, Q1s, MN1, MX1, CNT, A1, x_pad, avl,
                 Qx1, Qa1, Qb1, Qc1, qb1, lW1, lb1)
    stats = _bn_stats(h)
    act, AB2 = _bn_act(h, stats, bn_g, bn_b, W2, bias2)
    A2, B2 = AB2[:, :HID], AB2[:, HID:]
    S2, Q2s, MN2, MX2 = _seg2(dst, src, ea, eb, B2, C2)
    z = _combine(HID, NUM_CLASSES, S2, Q2s, MN2, MX2, CNT, A2, act, avl,
                 Qx2, Qa2, Qb2, Qc2, qb2, lW2, lb2)
    out = _logsm(z)
    return out[:N]
